# baseline (speedup 1.0000x reference)
import functools

import jax
import jax.numpy as jnp
from jax import lax
from jax.experimental import pallas as pl
from jax.experimental.pallas import tpu as pltpu
from jax.experimental.pallas import tpu_sc as plsc

VOCAB = 1000
EMBED = 128
HIDDEN = 768
NWORDS = 16384
MAXFEATS = 20

L = 16
NCORES = 2
NSUB = 16
NTILES = NCORES * NSUB
COL_SLICES = EMBED // L
WGROUPS = NTILES // COL_SLICES
WORDS_PER_TILE = NWORDS // WGROUPS
CHUNK_W = 256
NCHUNKS = WORDS_PER_TILE // CHUNK_W
GROUPS_PER_CHUNK = CHUNK_W // L
FSTRIDE = 21


def _pool_body(ids_hbm, mask_hbm, table_hbm, out_hbm, tbl_v,
               ids_v0, ids_v1, mask_v0, mask_v1, out_v0, out_v1,
               sem_in0, sem_in1, sem_out0, sem_out1):
    wid = lax.axis_index("s") * NCORES + lax.axis_index("c")
    cslice = wid % COL_SLICES
    wgroup = wid // COL_SLICES
    word_base = wgroup * WORDS_PER_TILE

    ids_bufs = [ids_v0, ids_v1]
    mask_bufs = [mask_v0, mask_v1]
    out_bufs = [out_v0, out_v1]
    sem_in = [sem_in0, sem_in1]
    sem_out = [sem_out0, sem_out1]

    pltpu.sync_copy(table_hbm.at[:, pl.ds(cslice * L, L)], tbl_v.at[:, pl.ds(0, L)])

    lanes = lax.iota(jnp.int32, 16)

    def in_copies(ci, buf):
        w0 = word_base + ci * CHUNK_W
        c1 = pltpu.make_async_copy(
            ids_hbm.at[pl.ds(w0, CHUNK_W), :], ids_bufs[buf], sem_in[buf]
        )
        c2 = pltpu.make_async_copy(
            mask_hbm.at[pl.ds(w0, CHUNK_W), :], mask_bufs[buf], sem_in[buf]
        )
        return c1, c2

    def make_group_body(ids_ref, mask_ref, out_ref):
        def group_body(g, carry):
            rows = g * L + lanes
            acc = [jnp.zeros((L,), jnp.float32) for _ in range(L)]
            cnt = jnp.zeros((L,), jnp.float32)
            for f in range(MAXFEATS):
                colf = jnp.full((L,), f, jnp.int32)
                ids_vec = plsc.load_gather(ids_ref, [rows, colf])
                m_vec = plsc.load_gather(mask_ref, [rows, colf])
                cnt = cnt + m_vec
                for d in range(L):
                    col = jnp.full((L,), d, jnp.int32)
                    gv = plsc.load_gather(tbl_v, [ids_vec, col])
                    acc[d] = acc[d] + m_vec * gv
            rcp = 1.0 / jnp.maximum(cnt, 1.0)
            for d in range(L):
                out_ref[d, pl.ds(g * L, L)] = acc[d] * rcp
            return carry

        return group_body

    c1, c2 = in_copies(0, 0)
    c1.start()
    c2.start()
    out_copies = [None, None]
    for ci in range(NCHUNKS):
        cur = ci % 2
        c1, c2 = in_copies(ci, cur)
        c1.wait()
        c2.wait()
        if ci + 1 < NCHUNKS:
            n1, n2 = in_copies(ci + 1, 1 - cur)
            n1.start()
            n2.start()
        if out_copies[cur] is not None:
            out_copies[cur].wait()
        lax.fori_loop(
            0, GROUPS_PER_CHUNK,
            make_group_body(ids_bufs[cur], mask_bufs[cur], out_bufs[cur]), 0,
        )
        w0 = word_base + ci * CHUNK_W
        oc = pltpu.make_async_copy(
            out_bufs[cur],
            out_hbm.at[pl.ds(cslice * L, L), pl.ds(w0, CHUNK_W)],
            sem_out[cur],
        )
        oc.start()
        out_copies[cur] = oc
    out_copies[0].wait()
    out_copies[1].wait()


@functools.partial(
    pl.kernel,
    out_type=jax.ShapeDtypeStruct((EMBED, NWORDS), jnp.float32),
    mesh=plsc.VectorSubcoreMesh(core_axis_name="c", subcore_axis_name="s"),
    scratch_types=[
        pltpu.VMEM((VOCAB, L + 1), jnp.float32),
        pltpu.VMEM((CHUNK_W, FSTRIDE), jnp.int32),
        pltpu.VMEM((CHUNK_W, FSTRIDE), jnp.int32),
        pltpu.VMEM((CHUNK_W, FSTRIDE), jnp.float32),
        pltpu.VMEM((CHUNK_W, FSTRIDE), jnp.float32),
        pltpu.VMEM((L, CHUNK_W), jnp.float32),
        pltpu.VMEM((L, CHUNK_W), jnp.float32),
        pltpu.SemaphoreType.DMA,
        pltpu.SemaphoreType.DMA,
        pltpu.SemaphoreType.DMA,
        pltpu.SemaphoreType.DMA,
    ],
    compiler_params=pltpu.CompilerParams(
        use_tc_tiling_on_sc=False, needs_layout_passes=False
    ),
)
def _pool_sc(ids_hbm, mask_hbm, table_hbm, out_hbm, *rest):
    _pool_body(ids_hbm, mask_hbm, table_hbm, out_hbm, *rest)


BN = 2048


def _mm_body(p_ref, w_ref, b_ref, o_ref):
    o_ref[...] = (
        jax.lax.dot_general(
            p_ref[...], w_ref[...],
            (((0,), (0,)), ((), ())),
            preferred_element_type=jnp.float32,
        )
        + b_ref[...]
    )


def _project(pooled_t, W, b):
    return pl.pallas_call(
        _mm_body,
        grid=(NWORDS // BN,),
        in_specs=[
            pl.BlockSpec((EMBED, BN), lambda i: (0, i)),
            pl.BlockSpec((EMBED, HIDDEN), lambda i: (0, 0)),
            pl.BlockSpec((1, HIDDEN), lambda i: (0, 0)),
        ],
        out_specs=pl.BlockSpec((BN, HIDDEN), lambda i: (i, 0)),
        out_shape=jax.ShapeDtypeStruct((NWORDS, HIDDEN), jnp.float32),
    )(pooled_t, W, b.reshape(1, HIDDEN))


def kernel(morph_ids, morph_mask, table, W, b):
    pad = ((0, 0), (0, FSTRIDE - MAXFEATS))
    ids2d = jnp.pad(morph_ids.reshape(NWORDS, MAXFEATS).astype(jnp.int32), pad)
    mask2d = jnp.pad(morph_mask.reshape(NWORDS, MAXFEATS), pad)
    pooled = _pool_sc(ids2d, mask2d, table)
    out = _project(pooled, W, b)
    return out.reshape(1, NWORDS, HIDDEN)


# speedup vs baseline: 1.3976x; 1.3976x over previous
import functools

import jax
import jax.numpy as jnp
from jax import lax
from jax.experimental import pallas as pl
from jax.experimental.pallas import tpu as pltpu
from jax.experimental.pallas import tpu_sc as plsc

VOCAB = 1000
EMBED = 128
HIDDEN = 768
NWORDS = 16384
MAXFEATS = 20

L = 16
NCORES = 2
NSUB = 16
NTILES = NCORES * NSUB
COL_SLICES = EMBED // L
WGROUPS = NTILES // COL_SLICES
WORDS_PER_TILE = NWORDS // WGROUPS
CHUNK_W = 256
NCHUNKS = WORDS_PER_TILE // CHUNK_W
GROUPS_PER_CHUNK = CHUNK_W // L
FSTRIDE = 21


def _pool_body(ids_hbm, mask_hbm, table_hbm, out_hbm, tbl_v,
               ids_v0, ids_v1, mask_v0, mask_v1, out_v0, out_v1,
               sem_in0, sem_in1, sem_out0, sem_out1):
    wid = lax.axis_index("s") * NCORES + lax.axis_index("c")
    cslice = wid % COL_SLICES
    wgroup = wid // COL_SLICES
    word_base = wgroup * WORDS_PER_TILE

    ids_bufs = [ids_v0, ids_v1]
    mask_bufs = [mask_v0, mask_v1]
    out_bufs = [out_v0, out_v1]
    sem_in = [sem_in0, sem_in1]
    sem_out = [sem_out0, sem_out1]

    pltpu.sync_copy(table_hbm.at[:, pl.ds(cslice * L, L)], tbl_v.at[:, pl.ds(0, L)])

    lanes = lax.iota(jnp.int32, 16)

    def in_copies(ci, buf):
        w0 = word_base + ci * CHUNK_W
        c1 = pltpu.make_async_copy(
            ids_hbm.at[:, pl.ds(w0, CHUNK_W)], ids_bufs[buf], sem_in[buf]
        )
        c2 = pltpu.make_async_copy(
            mask_hbm.at[:, pl.ds(w0, CHUNK_W)], mask_bufs[buf], sem_in[buf]
        )
        return c1, c2

    def make_group_body(ids_ref, mask_ref, out_ref):
        def group_body(g, carry):
            rows = g * L + lanes
            acc = [jnp.zeros((L,), jnp.float32) for _ in range(L)]
            cnt = jnp.zeros((L,), jnp.float32)
            for f in range(MAXFEATS):
                ids_vec = ids_ref[f, pl.ds(g * L, L)]
                m_vec = mask_ref[f, pl.ds(g * L, L)]
                cnt = cnt + m_vec
                for d in range(L):
                    col = jnp.full((L,), d, jnp.int32)
                    gv = plsc.load_gather(tbl_v, [ids_vec, col])
                    acc[d] = acc[d] + m_vec * gv
            rcp = 1.0 / jnp.maximum(cnt, 1.0)
            for d in range(L):
                col = jnp.full((L,), d, jnp.int32)
                plsc.store_scatter(out_ref, [rows, col], acc[d] * rcp)
            return carry

        return group_body

    c1, c2 = in_copies(0, 0)
    c1.start()
    c2.start()
    out_copies = [None, None]
    for ci in range(NCHUNKS):
        cur = ci % 2
        c1, c2 = in_copies(ci, cur)
        c1.wait()
        c2.wait()
        if ci + 1 < NCHUNKS:
            n1, n2 = in_copies(ci + 1, 1 - cur)
            n1.start()
            n2.start()
        if out_copies[cur] is not None:
            out_copies[cur].wait()
        lax.fori_loop(
            0, GROUPS_PER_CHUNK,
            make_group_body(ids_bufs[cur], mask_bufs[cur], out_bufs[cur]), 0,
        )
        w0 = word_base + ci * CHUNK_W
        oc = pltpu.make_async_copy(
            out_bufs[cur],
            out_hbm.at[pl.ds(w0, CHUNK_W), pl.ds(cslice * L, L)],
            sem_out[cur],
        )
        oc.start()
        out_copies[cur] = oc
    out_copies[0].wait()
    out_copies[1].wait()


@functools.partial(
    pl.kernel,
    out_type=jax.ShapeDtypeStruct((NWORDS, EMBED), jnp.float32),
    mesh=plsc.VectorSubcoreMesh(core_axis_name="c", subcore_axis_name="s"),
    scratch_types=[
        pltpu.VMEM((VOCAB, L + 1), jnp.float32),
        pltpu.VMEM((MAXFEATS, CHUNK_W), jnp.int32),
        pltpu.VMEM((MAXFEATS, CHUNK_W), jnp.int32),
        pltpu.VMEM((MAXFEATS, CHUNK_W), jnp.float32),
        pltpu.VMEM((MAXFEATS, CHUNK_W), jnp.float32),
        pltpu.VMEM((CHUNK_W, L), jnp.float32),
        pltpu.VMEM((CHUNK_W, L), jnp.float32),
        pltpu.SemaphoreType.DMA,
        pltpu.SemaphoreType.DMA,
        pltpu.SemaphoreType.DMA,
        pltpu.SemaphoreType.DMA,
    ],
    compiler_params=pltpu.CompilerParams(
        use_tc_tiling_on_sc=False, needs_layout_passes=False
    ),
)
def _pool_sc(ids_hbm, mask_hbm, table_hbm, out_hbm, *rest):
    _pool_body(ids_hbm, mask_hbm, table_hbm, out_hbm, *rest)


BN = 4096


def _mm_body(p_ref, w_ref, b_ref, o_ref):
    o_ref[...] = (
        jnp.dot(p_ref[...], w_ref[...], preferred_element_type=jnp.float32)
        + b_ref[...]
    )


def _project(pooled, W, b):
    return pl.pallas_call(
        _mm_body,
        grid=(NWORDS // BN,),
        in_specs=[
            pl.BlockSpec((BN, EMBED), lambda i: (i, 0)),
            pl.BlockSpec((EMBED, HIDDEN), lambda i: (0, 0)),
            pl.BlockSpec((1, HIDDEN), lambda i: (0, 0)),
        ],
        out_specs=pl.BlockSpec((BN, HIDDEN), lambda i: (i, 0)),
        out_shape=jax.ShapeDtypeStruct((NWORDS, HIDDEN), jnp.float32),
    )(pooled, W, b.reshape(1, HIDDEN))


def kernel(morph_ids, morph_mask, table, W, b):
    ids_t = morph_ids.reshape(NWORDS, MAXFEATS).astype(jnp.int32).T
    mask_t = morph_mask.reshape(NWORDS, MAXFEATS).T
    pooled = _pool_sc(ids_t, mask_t, table)
    out = _project(pooled, W, b)
    return out.reshape(1, NWORDS, HIDDEN)


# speedup vs baseline: 1.4330x; 1.0254x over previous
import functools

import jax
import jax.numpy as jnp
from jax import lax
from jax.experimental import pallas as pl
from jax.experimental.pallas import tpu as pltpu
from jax.experimental.pallas import tpu_sc as plsc

VOCAB = 1000
EMBED = 128
HIDDEN = 768
NWORDS = 16384
MAXFEATS = 20

L = 16
NCORES = 2
NSUB = 16
NTILES = NCORES * NSUB
COL_SLICES = EMBED // L
WGROUPS = NTILES // COL_SLICES
WORDS_PER_TILE = NWORDS // WGROUPS
CHUNK_W = 512
NCHUNKS = WORDS_PER_TILE // CHUNK_W
GROUPS_PER_CHUNK = CHUNK_W // L
FSTRIDE = 21


def _pool_body(ids_hbm, mask_hbm, table_hbm, out_hbm, tbl_v,
               ids_v0, ids_v1, mask_v0, mask_v1, out_v0, out_v1,
               sem_in0, sem_in1, sem_out0, sem_out1):
    wid = lax.axis_index("s") * NCORES + lax.axis_index("c")
    cslice = wid % COL_SLICES
    wgroup = wid // COL_SLICES
    word_base = wgroup * WORDS_PER_TILE

    ids_bufs = [ids_v0, ids_v1]
    mask_bufs = [mask_v0, mask_v1]
    out_bufs = [out_v0, out_v1]
    sem_in = [sem_in0, sem_in1]
    sem_out = [sem_out0, sem_out1]

    pltpu.sync_copy(table_hbm.at[:, pl.ds(cslice * L, L)], tbl_v.at[:, pl.ds(0, L)])

    lanes = lax.iota(jnp.int32, 16)

    def in_copies(ci, buf):
        w0 = word_base + ci * CHUNK_W
        c1 = pltpu.make_async_copy(
            ids_hbm.at[:, pl.ds(w0, CHUNK_W)], ids_bufs[buf], sem_in[buf]
        )
        c2 = pltpu.make_async_copy(
            mask_hbm.at[:, pl.ds(w0, CHUNK_W)], mask_bufs[buf], sem_in[buf]
        )
        return c1, c2

    def make_group_body(ids_ref, mask_ref, out_ref):
        def group_body(g, carry):
            rows = g * L + lanes
            acc = [jnp.zeros((L,), jnp.float32) for _ in range(L)]
            cnt = jnp.zeros((L,), jnp.float32)
            for f in range(MAXFEATS):
                ids_vec = ids_ref[f, pl.ds(g * L, L)]
                m_vec = mask_ref[f, pl.ds(g * L, L)]
                cnt = cnt + m_vec
                for d in range(L):
                    col = jnp.full((L,), d, jnp.int32)
                    gv = plsc.load_gather(tbl_v, [ids_vec, col])
                    acc[d] = acc[d] + m_vec * gv
            rcp = 1.0 / jnp.maximum(cnt, 1.0)
            for d in range(L):
                col = jnp.full((L,), d, jnp.int32)
                plsc.store_scatter(out_ref, [rows, col], acc[d] * rcp)
            return carry

        return group_body

    c1, c2 = in_copies(0, 0)
    c1.start()
    c2.start()
    out_copies = [None, None]
    for ci in range(NCHUNKS):
        cur = ci % 2
        c1, c2 = in_copies(ci, cur)
        c1.wait()
        c2.wait()
        if ci + 1 < NCHUNKS:
            n1, n2 = in_copies(ci + 1, 1 - cur)
            n1.start()
            n2.start()
        if out_copies[cur] is not None:
            out_copies[cur].wait()
        lax.fori_loop(
            0, GROUPS_PER_CHUNK,
            make_group_body(ids_bufs[cur], mask_bufs[cur], out_bufs[cur]), 0,
        )
        w0 = word_base + ci * CHUNK_W
        oc = pltpu.make_async_copy(
            out_bufs[cur],
            out_hbm.at[pl.ds(w0, CHUNK_W), pl.ds(cslice * L, L)],
            sem_out[cur],
        )
        oc.start()
        out_copies[cur] = oc
    out_copies[0].wait()
    out_copies[1].wait()


@functools.partial(
    pl.kernel,
    out_type=jax.ShapeDtypeStruct((NWORDS, EMBED), jnp.float32),
    mesh=plsc.VectorSubcoreMesh(core_axis_name="c", subcore_axis_name="s"),
    scratch_types=[
        pltpu.VMEM((VOCAB, L + 1), jnp.float32),
        pltpu.VMEM((MAXFEATS, CHUNK_W), jnp.int32),
        pltpu.VMEM((MAXFEATS, CHUNK_W), jnp.int32),
        pltpu.VMEM((MAXFEATS, CHUNK_W), jnp.float32),
        pltpu.VMEM((MAXFEATS, CHUNK_W), jnp.float32),
        pltpu.VMEM((CHUNK_W, L), jnp.float32),
        pltpu.VMEM((CHUNK_W, L), jnp.float32),
        pltpu.SemaphoreType.DMA,
        pltpu.SemaphoreType.DMA,
        pltpu.SemaphoreType.DMA,
        pltpu.SemaphoreType.DMA,
    ],
    compiler_params=pltpu.CompilerParams(
        use_tc_tiling_on_sc=False, needs_layout_passes=False
    ),
)
def _pool_sc(ids_hbm, mask_hbm, table_hbm, out_hbm, *rest):
    _pool_body(ids_hbm, mask_hbm, table_hbm, out_hbm, *rest)


BN = 4096


def _mm_body(p_ref, w_ref, b_ref, o_ref):
    o_ref[...] = (
        jnp.dot(p_ref[...], w_ref[...], preferred_element_type=jnp.float32)
        + b_ref[...]
    )


def _project(pooled, W, b):
    return pl.pallas_call(
        _mm_body,
        grid=(NWORDS // BN,),
        in_specs=[
            pl.BlockSpec((BN, EMBED), lambda i: (i, 0)),
            pl.BlockSpec((EMBED, HIDDEN), lambda i: (0, 0)),
            pl.BlockSpec((1, HIDDEN), lambda i: (0, 0)),
        ],
        out_specs=pl.BlockSpec((BN, HIDDEN), lambda i: (i, 0)),
        out_shape=jax.ShapeDtypeStruct((NWORDS, HIDDEN), jnp.float32),
    )(pooled, W, b.reshape(1, HIDDEN))


def kernel(morph_ids, morph_mask, table, W, b):
    ids_t = morph_ids.reshape(NWORDS, MAXFEATS).astype(jnp.int32).T
    mask_t = morph_mask.reshape(NWORDS, MAXFEATS).T
    pooled = _pool_sc(ids_t, mask_t, table)
    out = _project(pooled, W, b)
    return out.reshape(1, NWORDS, HIDDEN)


# speedup vs baseline: 1.4332x; 1.0001x over previous
import functools

import jax
import jax.numpy as jnp
from jax import lax
from jax.experimental import pallas as pl
from jax.experimental.pallas import tpu as pltpu
from jax.experimental.pallas import tpu_sc as plsc

VOCAB = 1000
EMBED = 128
HIDDEN = 768
NWORDS = 16384
MAXFEATS = 20

L = 16
NCORES = 2
NSUB = 16
NTILES = NCORES * NSUB
COL_SLICES = EMBED // L
WGROUPS = NTILES // COL_SLICES
WORDS_PER_TILE = NWORDS // WGROUPS
CHUNK_W = 512
NCHUNKS = WORDS_PER_TILE // CHUNK_W
GROUPS_PER_CHUNK = CHUNK_W // L


def _pool_body(ids_hbm, mask_hbm, table_hbm, out_hbm, tbl_v,
               ids_v0, ids_v1, mask_v0, mask_v1, out_v0, out_v1,
               sem_in0, sem_in1, sem_out0, sem_out1):
    wid = lax.axis_index("s") * NCORES + lax.axis_index("c")
    cslice = wid % COL_SLICES
    wgroup = wid // COL_SLICES
    word_base = wgroup * WORDS_PER_TILE

    ids_bufs = [ids_v0, ids_v1]
    mask_bufs = [mask_v0, mask_v1]
    out_bufs = [out_v0, out_v1]
    sem_in = [sem_in0, sem_in1]
    sem_out = [sem_out0, sem_out1]

    pltpu.sync_copy(table_hbm.at[:, pl.ds(cslice * L, L)], tbl_v.at[:, pl.ds(0, L)])

    lanes = lax.iota(jnp.int32, 16)

    def in_copies(ci, buf):
        w0 = word_base + ci * CHUNK_W
        c1 = pltpu.make_async_copy(
            ids_hbm.at[:, pl.ds(w0, CHUNK_W)], ids_bufs[buf], sem_in[buf]
        )
        c2 = pltpu.make_async_copy(
            mask_hbm.at[:, pl.ds(w0, CHUNK_W)], mask_bufs[buf], sem_in[buf]
        )
        return c1, c2

    def make_group_body(ids_ref, mask_ref, out_ref):
        def group_body(g, carry):
            rows = g * L + lanes
            acc = [jnp.zeros((L,), jnp.float32) for _ in range(L)]
            cnt = jnp.zeros((L,), jnp.float32)
            for f in range(MAXFEATS):
                ids_vec = ids_ref[f, pl.ds(g * L, L)]
                m_vec = mask_ref[f, pl.ds(g * L, L)]
                cnt = cnt + m_vec
                for d in range(L):
                    col = jnp.full((L,), d, jnp.int32)
                    gv = plsc.load_gather(tbl_v, [ids_vec, col])
                    acc[d] = acc[d] + m_vec * gv
            rcp = 1.0 / jnp.maximum(cnt, 1.0)
            for d in range(L):
                col = jnp.full((L,), d, jnp.int32)
                plsc.store_scatter(out_ref, [rows, col], acc[d] * rcp)
            return carry

        return group_body

    c1, c2 = in_copies(0, 0)
    c1.start()
    c2.start()
    out_copies = [None, None]
    for ci in range(NCHUNKS):
        cur = ci % 2
        c1, c2 = in_copies(ci, cur)
        c1.wait()
        c2.wait()
        if ci + 1 < NCHUNKS:
            n1, n2 = in_copies(ci + 1, 1 - cur)
            n1.start()
            n2.start()
        if out_copies[cur] is not None:
            out_copies[cur].wait()
        lax.fori_loop(
            0, GROUPS_PER_CHUNK,
            make_group_body(ids_bufs[cur], mask_bufs[cur], out_bufs[cur]), 0,
        )
        w0 = word_base + ci * CHUNK_W
        oc = pltpu.make_async_copy(
            out_bufs[cur],
            out_hbm.at[pl.ds(w0, CHUNK_W), pl.ds(cslice * L, L)],
            sem_out[cur],
        )
        oc.start()
        out_copies[cur] = oc
    out_copies[0].wait()
    out_copies[1].wait()


@functools.partial(
    pl.kernel,
    out_type=jax.ShapeDtypeStruct((NWORDS, EMBED), jnp.float32),
    mesh=plsc.VectorSubcoreMesh(core_axis_name="c", subcore_axis_name="s"),
    scratch_types=[
        pltpu.VMEM((VOCAB, L + 1), jnp.float32),
        pltpu.VMEM((MAXFEATS, CHUNK_W), jnp.int32),
        pltpu.VMEM((MAXFEATS, CHUNK_W), jnp.int32),
        pltpu.VMEM((MAXFEATS, CHUNK_W), jnp.float32),
        pltpu.VMEM((MAXFEATS, CHUNK_W), jnp.float32),
        pltpu.VMEM((CHUNK_W, L), jnp.float32),
        pltpu.VMEM((CHUNK_W, L), jnp.float32),
        pltpu.SemaphoreType.DMA,
        pltpu.SemaphoreType.DMA,
        pltpu.SemaphoreType.DMA,
        pltpu.SemaphoreType.DMA,
    ],
    compiler_params=pltpu.CompilerParams(
        use_tc_tiling_on_sc=False, needs_layout_passes=False
    ),
)
def _pool_sc(ids_hbm, mask_hbm, table_hbm, out_hbm, *rest):
    _pool_body(ids_hbm, mask_hbm, table_hbm, out_hbm, *rest)


BN = 4096


def _mm_body(p_ref, w_ref, b_ref, o_ref):
    o_ref[...] = (
        jnp.dot(p_ref[...], w_ref[...], preferred_element_type=jnp.float32)
        + b_ref[...]
    )


def _project(pooled, W, b):
    return pl.pallas_call(
        _mm_body,
        grid=(NWORDS // BN,),
        in_specs=[
            pl.BlockSpec((BN, EMBED), lambda i: (i, 0)),
            pl.BlockSpec((EMBED, HIDDEN), lambda i: (0, 0)),
            pl.BlockSpec((1, HIDDEN), lambda i: (0, 0)),
        ],
        out_specs=pl.BlockSpec((BN, HIDDEN), lambda i: (i, 0)),
        out_shape=jax.ShapeDtypeStruct((NWORDS, HIDDEN), jnp.float32),
    )(pooled, W, b.reshape(1, HIDDEN))


def kernel(morph_ids, morph_mask, table, W, b):
    ids_t = morph_ids.reshape(NWORDS, MAXFEATS).astype(jnp.int32).T
    mask_t = morph_mask.reshape(NWORDS, MAXFEATS).T
    pooled = _pool_sc(ids_t, mask_t, table)
    out = _project(pooled, W, b)
    return out.reshape(1, NWORDS, HIDDEN)
